# Initial kernel scaffold; baseline (speedup 1.0000x reference)
import jax
import jax.numpy as jnp
from jax.experimental import pallas as pl


def kernel(x, edge_attr, W1, root1, bias1, g1, be1, W2, root2, bias2, g2, be2, W3, root3, bias3, g3, be3, W4, root4, bias4, g4, be4, Wm, rootm, biasm, gm, bem, lin1, lin2, edge_index, pos):
    raise NotImplementedError("write your pallas kernel here")


# speedup vs baseline: 1.0666x; 1.0666x over previous
import functools

import jax
import jax.numpy as jnp
from jax.experimental import pallas as pl

KS = 4
K3 = KS ** 3


def _basis(edge_attr):
    u = jnp.clip(edge_attr, 0.0, 1.0) * (KS - 1)
    base = jnp.floor(u)
    frac = u - base
    base_i = base.astype(jnp.int32)
    ws, kidxs = [], []
    for o0 in (0, 1):
        for o1 in (0, 1):
            for o2 in (0, 1):
                o = (o0, o1, o2)
                w = jnp.ones(edge_attr.shape[:1], jnp.float32)
                kidx = jnp.zeros(edge_attr.shape[:1], jnp.int32)
                mult = 1
                for d in range(3):
                    fd = frac[:, d]
                    w = w * (fd if o[d] == 1 else (1.0 - fd))
                    kidx = kidx + jnp.clip(base_i[:, d] + o[d], 0, KS - 1) * mult
                    mult = mult * KS
                ws.append(w)
                kidxs.append(kidx)
    return jnp.stack(ws, axis=1), jnp.stack(kidxs, axis=1)


def _spline_layer(x, src, dst, w8, k8, deg, W, root, bias):
    N = x.shape[0]
    K, in_c, out_c = W.shape
    x_src = x[src]
    A = jnp.zeros((N * K, in_c), x.dtype)
    for c in range(8):
        A = A.at[dst * K + k8[:, c]].add(x_src * w8[:, c][:, None])
    agg = A.reshape(N, K * in_c) @ W.reshape(K * in_c, out_c)
    agg = agg / deg[:, None]
    return agg + x @ root + bias


def _block(x, src, dst, w8, k8, deg, W, root, bias, gamma, beta):
    h = jax.nn.elu(_spline_layer(x, src, dst, w8, k8, deg, W, root, bias))
    m = h.mean(axis=0)
    v = h.var(axis=0)
    return gamma * (h - m) / jnp.sqrt(v + 1e-5) + beta


def _head_kernel(feat_ref, lin1_ref, lin2_ref, out_ref):
    t = jnp.dot(feat_ref[...], lin1_ref[...], preferred_element_type=jnp.float32)
    out_ref[...] = jnp.dot(t, lin2_ref[...], preferred_element_type=jnp.float32)


def _head(feat, lin1, lin2):
    return pl.pallas_call(
        _head_kernel,
        out_shape=jax.ShapeDtypeStruct((1, lin2.shape[1]), jnp.float32),
    )(feat, lin1, lin2)


def kernel(x, edge_attr, W1, root1, bias1, g1, be1, W2, root2, bias2, g2, be2,
           W3, root3, bias3, g3, be3, W4, root4, bias4, g4, be4,
           Wm, rootm, biasm, gm, bem, lin1, lin2, edge_index, pos):
    src = edge_index[0]
    dst = edge_index[1]
    N = x.shape[0]
    w8, k8 = _basis(edge_attr)
    deg = jax.ops.segment_sum(jnp.ones((dst.shape[0],), jnp.float32), dst,
                              num_segments=N)
    deg = jnp.maximum(deg, 1.0)

    h = _block(x, src, dst, w8, k8, deg, W1, root1, bias1, g1, be1)
    h = _block(h, src, dst, w8, k8, deg, W2, root2, bias2, g2, be2)
    h = _block(h, src, dst, w8, k8, deg, W3, root3, bias3, g3, be3)
    h = _block(h, src, dst, w8, k8, deg, W4, root4, bias4, g4, be4)

    cx = jnp.clip(jnp.floor(pos[:, 0] / 120.0), 0, 1).astype(jnp.int32)
    cy = jnp.clip(jnp.floor(pos[:, 1] / 90.0), 0, 1).astype(jnp.int32)
    cluster = cy * 2 + cx
    pooled = jax.ops.segment_max(h, cluster, num_segments=4)
    feat = pooled.reshape(1, 4 * 512)
    labels = _head(feat, lin1, lin2)

    xm = jnp.zeros((N, 1), jnp.float32)
    return labels, xm


# speedup vs baseline: 3.2604x; 3.0567x over previous
import functools

import jax
import jax.numpy as jnp
from jax import lax
from jax.experimental import pallas as pl
from jax.experimental.pallas import tpu as pltpu
from jax.experimental.pallas import tpu_sc as plsc

KS = 4
K3 = KS ** 3
TN = 4
C = 64
NW = 32


def _basis(edge_attr):
    u = jnp.clip(edge_attr, 0.0, 1.0) * (KS - 1)
    base = jnp.floor(u)
    frac = u - base
    base_i = base.astype(jnp.int32)
    ws, kidxs = [], []
    for o0 in (0, 1):
        for o1 in (0, 1):
            for o2 in (0, 1):
                o = (o0, o1, o2)
                w = jnp.ones(edge_attr.shape[:1], jnp.float32)
                kidx = jnp.zeros(edge_attr.shape[:1], jnp.int32)
                mult = 1
                for d in range(3):
                    fd = frac[:, d]
                    w = w * (fd if o[d] == 1 else (1.0 - fd))
                    kidx = kidx + jnp.clip(base_i[:, d] + o[d], 0, KS - 1) * mult
                    mult = mult * KS
                ws.append(w)
                kidxs.append(kidx)
    return jnp.stack(ws, axis=1), jnp.stack(kidxs, axis=1)


def _sc_layer(x, src_s, w8f, r8f, starts_p, ic):
    n_nodes = x.shape[0]
    G = n_nodes // TN
    GP = starts_p.shape[0]
    J = ic // 16
    ROWS = TN * 64
    mesh = plsc.VectorSubcoreMesh(core_axis_name="c", subcore_axis_name="s")

    @functools.partial(
        pl.kernel,
        out_type=jax.ShapeDtypeStruct((G * ROWS * ic,), jnp.float32),
        mesh=mesh,
        scratch_types=[
            pltpu.VMEM((GP,), jnp.int32),
            pltpu.VMEM((C,), jnp.int32),
            pltpu.VMEM((C * 8 + 16,), jnp.float32),
            pltpu.VMEM((C * 8 + 16,), jnp.int32),
            pltpu.VMEM((C, ic), jnp.float32),
            pltpu.VMEM((ROWS * ic,), jnp.float32),
            pltpu.SemaphoreType.DMA,
        ],
        compiler_params=pltpu.CompilerParams(use_tc_tiling_on_sc=False),
    )
    def k(x_hbm, src_hbm, w_hbm, r_hbm, starts_hbm, a_hbm,
          starts_v, src_v, w_v, r_v, rows_v, a_v, sem):
        wid = lax.axis_index("s") * 2 + lax.axis_index("c")
        pltpu.sync_copy(starts_hbm, starts_v)
        ntw = (jnp.int32(G) - wid + (NW - 1)) // NW

        def tile_body(i, _):
            t = wid + i * NW

            def zbody(z, _):
                for u in range(8):
                    a_v[pl.ds((z * 8 + u) * 16, 16)] = jnp.zeros((16,),
                                                                 jnp.float32)
                return 0

            lax.fori_loop(0, ROWS * J // 8, zbody, 0)
            sv = starts_v[pl.ds(t, 16)]
            s = sv[0]
            e = sv[1]
            cb0 = s // C
            cb1 = (e + (C - 1)) // C

            def chunk_body(cb, _):
                pltpu.sync_copy(src_hbm.at[pl.ds(cb * C, C)], src_v)
                pltpu.sync_copy(w_hbm.at[pl.ds(cb * C * 8, C * 8)],
                                w_v.at[pl.ds(0, C * 8)])
                pltpu.sync_copy(r_hbm.at[pl.ds(cb * C * 8, C * 8)],
                                r_v.at[pl.ds(0, C * 8)])
                pltpu.async_copy(x_hbm.at[src_v], rows_v, sem).wait()
                elo = jnp.maximum(s - cb * C, 0)
                ehi = jnp.minimum(e - cb * C, C)

                def edge_body(el, _):
                    segs = [rows_v[el, pl.ds(16 * j, 16)] for j in range(J)]
                    wv = w_v[pl.ds(el * 8, 16)]
                    rv = r_v[pl.ds(el * 8, 16)]
                    for c in range(8):
                        w = wv[c]
                        base = rv[c] * ic
                        for j in range(J):
                            plsc.addupdate(a_v.at[pl.ds(base + 16 * j, 16)],
                                           w * segs[j])
                    return 0

                lax.fori_loop(elo, ehi, edge_body, 0)
                return 0

            lax.fori_loop(cb0, cb1, chunk_body, 0)
            pltpu.sync_copy(a_v, a_hbm.at[pl.ds(t * ROWS * ic, ROWS * ic)])
            return 0

        lax.fori_loop(0, ntw, tile_body, 0)

    return k(x, src_s, w8f, r8f, starts_p)


def _mm_layer(a_flat, W, x, root, bias, invdeg):
    n, ic = x.shape
    kd = W.shape[0] * W.shape[1]
    oc = W.shape[2]
    a2 = a_flat.reshape(n, kd)
    wr = W.reshape(kd, oc)
    kt = min(kd, 2048)
    nk = kd // kt
    nt = 1000
    ni = n // nt

    def body(a_ref, w_ref, x_ref, root_ref, bias_ref, invdeg_ref, out_ref,
             acc_ref):
        kk = pl.program_id(1)

        @pl.when(kk == 0)
        def _():
            acc_ref[...] = jnp.zeros_like(acc_ref)

        acc_ref[...] += jnp.dot(a_ref[...], w_ref[...],
                                preferred_element_type=jnp.float32)

        @pl.when(kk == nk - 1)
        def _():
            h = (acc_ref[...] * invdeg_ref[...]
                 + jnp.dot(x_ref[...], root_ref[...],
                           preferred_element_type=jnp.float32)
                 + bias_ref[...])
            out_ref[...] = jnp.where(h > 0, h, jnp.exp(h) - 1.0)

    return pl.pallas_call(
        body,
        grid=(ni, nk),
        in_specs=[
            pl.BlockSpec((nt, kt), lambda i, k: (i, k)),
            pl.BlockSpec((kt, oc), lambda i, k: (k, 0)),
            pl.BlockSpec((nt, ic), lambda i, k: (i, 0)),
            pl.BlockSpec((ic, oc), lambda i, k: (0, 0)),
            pl.BlockSpec((1, oc), lambda i, k: (0, 0)),
            pl.BlockSpec((nt, 1), lambda i, k: (i, 0)),
        ],
        out_specs=pl.BlockSpec((nt, oc), lambda i, k: (i, 0)),
        out_shape=jax.ShapeDtypeStruct((n, oc), jnp.float32),
        scratch_shapes=[pltpu.VMEM((nt, oc), jnp.float32)],
    )(a2, wr, x, root.reshape(ic, oc), bias.reshape(1, oc), invdeg)


def _bn(h, gamma, beta):
    n, oc = h.shape
    ct = min(oc, 128)

    def body(h_ref, g_ref, b_ref, out_ref):
        hb = h_ref[...]
        m = jnp.mean(hb, axis=0, keepdims=True)
        v = jnp.mean((hb - m) ** 2, axis=0, keepdims=True)
        out_ref[...] = g_ref[...] * (hb - m) / jnp.sqrt(v + 1e-5) + b_ref[...]

    return pl.pallas_call(
        body,
        grid=(oc // ct,),
        in_specs=[
            pl.BlockSpec((n, ct), lambda j: (0, j)),
            pl.BlockSpec((1, ct), lambda j: (0, j)),
            pl.BlockSpec((1, ct), lambda j: (0, j)),
        ],
        out_specs=pl.BlockSpec((n, ct), lambda j: (0, j)),
        out_shape=jax.ShapeDtypeStruct((n, oc), jnp.float32),
    )(h, gamma.reshape(1, oc), beta.reshape(1, oc))


def _pool_max(h, cluster):
    n, oc = h.shape
    ct = 128

    def body(h_ref, cl_ref, out_ref):
        hb = h_ref[...]
        cl = cl_ref[...]
        rows = [jnp.max(jnp.where(cl == c, hb, -jnp.inf), axis=0,
                        keepdims=True) for c in range(4)]
        out_ref[...] = jnp.concatenate(rows + rows, axis=0)

    return pl.pallas_call(
        body,
        grid=(oc // ct,),
        in_specs=[
            pl.BlockSpec((n, ct), lambda j: (0, j)),
            pl.BlockSpec((n, 1), lambda j: (0, 0)),
        ],
        out_specs=pl.BlockSpec((8, ct), lambda j: (0, j)),
        out_shape=jax.ShapeDtypeStruct((8, oc), jnp.float32),
    )(h, cluster.reshape(n, 1))


def _head(feat, lin1, lin2):
    def body(feat_ref, l1_ref, l2_ref, out_ref):
        t = jnp.dot(feat_ref[...], l1_ref[...],
                    preferred_element_type=jnp.float32)
        out_ref[...] = jnp.dot(t, l2_ref[...],
                               preferred_element_type=jnp.float32)

    return pl.pallas_call(
        body,
        out_shape=jax.ShapeDtypeStruct((1, lin2.shape[1]), jnp.float32),
    )(feat, lin1, lin2)


def kernel(x, edge_attr, W1, root1, bias1, g1, be1, W2, root2, bias2, g2, be2,
           W3, root3, bias3, g3, be3, W4, root4, bias4, g4, be4,
           Wm, rootm, biasm, gm, bem, lin1, lin2, edge_index, pos):
    src = edge_index[0]
    dst = edge_index[1]
    n = x.shape[0]
    n_edges = src.shape[0]
    G = n // TN

    w8, k8 = _basis(edge_attr)
    perm = jnp.argsort(dst)
    src_s = src[perm].astype(jnp.int32)
    dst_s = dst[perm].astype(jnp.int32)
    w8f = w8[perm].reshape(-1)
    r8f = ((dst_s % TN)[:, None] * 64 + k8[perm]).reshape(-1).astype(jnp.int32)
    starts = jnp.searchsorted(dst_s, jnp.arange(G + 1, dtype=jnp.int32) * TN
                              ).astype(jnp.int32)
    gp = ((G + 16 + 7) // 8) * 8
    starts_p = jnp.concatenate(
        [starts, jnp.full((gp - G - 1,), n_edges, jnp.int32)])

    node_b = jnp.searchsorted(dst_s, jnp.arange(n + 1, dtype=jnp.int32))
    deg = (node_b[1:] - node_b[:-1]).astype(jnp.float32).reshape(n, 1)
    invdeg = 1.0 / jnp.maximum(deg, 1.0)

    x16 = jnp.concatenate([x, jnp.zeros((n, 15), jnp.float32)], axis=1)
    a1p = _sc_layer(x16, src_s, w8f, r8f, starts_p, 16)
    a1 = a1p.reshape(n * 64, 16)[:, 0]
    h = _mm_layer(a1, W1, x, root1, bias1, invdeg)
    h = _bn(h, g1, be1)
    for (W, root, bias, gm_, be_) in ((W2, root2, bias2, g2, be2),
                                      (W3, root3, bias3, g3, be3),
                                      (W4, root4, bias4, g4, be4)):
        a = _sc_layer(h, src_s, w8f, r8f, starts_p, h.shape[1])
        h = _mm_layer(a, W, h, root, bias, invdeg)
        h = _bn(h, gm_, be_)

    cx = jnp.clip(jnp.floor(pos[:, 0] / 120.0), 0, 1).astype(jnp.int32)
    cy = jnp.clip(jnp.floor(pos[:, 1] / 90.0), 0, 1).astype(jnp.int32)
    cluster = cy * 2 + cx
    pooled = _pool_max(h, cluster)
    feat = pooled[:4].reshape(1, 4 * 512)
    labels = _head(feat, lin1, lin2)

    xm = jnp.zeros((n, 1), jnp.float32)
    return labels, xm


# speedup vs baseline: 3.8536x; 1.1819x over previous
import functools

import jax
import jax.numpy as jnp
from jax import lax
from jax.experimental import pallas as pl
from jax.experimental.pallas import tpu as pltpu
from jax.experimental.pallas import tpu_sc as plsc

KS = 4
K3 = KS ** 3
TN = 4
C = 64
NW = 32


def _basis(edge_attr):
    u = jnp.clip(edge_attr, 0.0, 1.0) * (KS - 1)
    base = jnp.floor(u)
    frac = u - base
    base_i = base.astype(jnp.int32)
    ws, kidxs = [], []
    for o0 in (0, 1):
        for o1 in (0, 1):
            for o2 in (0, 1):
                o = (o0, o1, o2)
                w = jnp.ones(edge_attr.shape[:1], jnp.float32)
                kidx = jnp.zeros(edge_attr.shape[:1], jnp.int32)
                mult = 1
                for d in range(3):
                    fd = frac[:, d]
                    w = w * (fd if o[d] == 1 else (1.0 - fd))
                    kidx = kidx + jnp.clip(base_i[:, d] + o[d], 0, KS - 1) * mult
                    mult = mult * KS
                ws.append(w)
                kidxs.append(kidx)
    return jnp.stack(ws, axis=1), jnp.stack(kidxs, axis=1)


def _sc_layer(x, src_s, w8f, r8f, starts_p, ic):
    n_nodes = x.shape[0]
    G = n_nodes // TN
    GP = starts_p.shape[0]
    J = ic // 16
    ROWS = TN * 64
    mesh = plsc.VectorSubcoreMesh(core_axis_name="c", subcore_axis_name="s")

    @functools.partial(
        pl.kernel,
        out_type=jax.ShapeDtypeStruct((G * ROWS * ic,), jnp.float32),
        mesh=mesh,
        scratch_types=[
            pltpu.VMEM((GP,), jnp.int32),
            pltpu.VMEM((C,), jnp.int32),
            pltpu.VMEM((C * 8 + 16,), jnp.float32),
            pltpu.VMEM((C * 8 + 16,), jnp.int32),
            pltpu.VMEM((C, ic), jnp.float32),
            pltpu.VMEM((ROWS * ic,), jnp.float32),
            pltpu.SemaphoreType.DMA,
        ],
        compiler_params=pltpu.CompilerParams(use_tc_tiling_on_sc=False),
    )
    def k(x_hbm, src_hbm, w_hbm, r_hbm, starts_hbm, a_hbm,
          starts_v, src_v, w_v, r_v, rows_v, a_v, sem):
        wid = lax.axis_index("s") * 2 + lax.axis_index("c")
        pltpu.sync_copy(starts_hbm, starts_v)
        ntw = (jnp.int32(G) - wid + (NW - 1)) // NW

        def tile_body(i, _):
            t = wid + i * NW

            def zbody(z, _):
                for u in range(8):
                    a_v[pl.ds((z * 8 + u) * 16, 16)] = jnp.zeros((16,),
                                                                 jnp.float32)
                return 0

            lax.fori_loop(0, ROWS * J // 8, zbody, 0)
            sv = starts_v[pl.ds(t, 16)]
            s = sv[0]
            e = sv[1]
            cb0 = s // C
            cb1 = (e + (C - 1)) // C

            def chunk_body(cb, _):
                pltpu.sync_copy(src_hbm.at[pl.ds(cb * C, C)], src_v)
                pltpu.sync_copy(w_hbm.at[pl.ds(cb * C * 8, C * 8)],
                                w_v.at[pl.ds(0, C * 8)])
                pltpu.sync_copy(r_hbm.at[pl.ds(cb * C * 8, C * 8)],
                                r_v.at[pl.ds(0, C * 8)])
                pltpu.async_copy(x_hbm.at[src_v], rows_v, sem).wait()
                elo = jnp.maximum(s - cb * C, 0)
                ehi = jnp.minimum(e - cb * C, C)

                def edge_body(el, _):
                    segs = [rows_v[el, pl.ds(16 * j, 16)] for j in range(J)]
                    wv = w_v[pl.ds(el * 8, 16)]
                    rv = r_v[pl.ds(el * 8, 16)]
                    for c in range(8):
                        w = wv[c]
                        base = rv[c] * ic
                        for j in range(J):
                            plsc.addupdate(a_v.at[pl.ds(base + 16 * j, 16)],
                                           w * segs[j])
                    return 0

                lax.fori_loop(elo, ehi, edge_body, 0)
                return 0

            lax.fori_loop(cb0, cb1, chunk_body, 0)
            pltpu.sync_copy(a_v, a_hbm.at[pl.ds(t * ROWS * ic, ROWS * ic)])
            return 0

        lax.fori_loop(0, ntw, tile_body, 0)

    return k(x, src_s, w8f, r8f, starts_p)


def _mm_layer(a_flat, W, x, root, bias, invdeg):
    n, ic = x.shape
    kd = W.shape[0] * W.shape[1]
    oc = W.shape[2]
    a2 = a_flat.reshape(n, kd)
    wr = W.reshape(kd, oc)
    kt = min(kd, 2048)
    nk = kd // kt
    nt = 1000
    ni = n // nt

    def body(a_ref, w_ref, x_ref, root_ref, bias_ref, invdeg_ref, out_ref,
             acc_ref):
        kk = pl.program_id(1)

        @pl.when(kk == 0)
        def _():
            acc_ref[...] = jnp.zeros_like(acc_ref)

        acc_ref[...] += jnp.dot(a_ref[...], w_ref[...],
                                preferred_element_type=jnp.float32)

        @pl.when(kk == nk - 1)
        def _():
            h = (acc_ref[...] * invdeg_ref[...]
                 + jnp.dot(x_ref[...], root_ref[...],
                           preferred_element_type=jnp.float32)
                 + bias_ref[...])
            out_ref[...] = jnp.where(h > 0, h, jnp.exp(h) - 1.0)

    return pl.pallas_call(
        body,
        grid=(ni, nk),
        in_specs=[
            pl.BlockSpec((nt, kt), lambda i, k: (i, k)),
            pl.BlockSpec((kt, oc), lambda i, k: (k, 0)),
            pl.BlockSpec((nt, ic), lambda i, k: (i, 0)),
            pl.BlockSpec((ic, oc), lambda i, k: (0, 0)),
            pl.BlockSpec((1, oc), lambda i, k: (0, 0)),
            pl.BlockSpec((nt, 1), lambda i, k: (i, 0)),
        ],
        out_specs=pl.BlockSpec((nt, oc), lambda i, k: (i, 0)),
        out_shape=jax.ShapeDtypeStruct((n, oc), jnp.float32),
        scratch_shapes=[pltpu.VMEM((nt, oc), jnp.float32)],
    )(a2, wr, x, root.reshape(ic, oc), bias.reshape(1, oc), invdeg)


def _bn(h, gamma, beta):
    n, oc = h.shape
    ct = min(oc, 128)

    def body(h_ref, g_ref, b_ref, out_ref):
        hb = h_ref[...]
        m = jnp.mean(hb, axis=0, keepdims=True)
        v = jnp.mean((hb - m) ** 2, axis=0, keepdims=True)
        out_ref[...] = g_ref[...] * (hb - m) / jnp.sqrt(v + 1e-5) + b_ref[...]

    return pl.pallas_call(
        body,
        grid=(oc // ct,),
        in_specs=[
            pl.BlockSpec((n, ct), lambda j: (0, j)),
            pl.BlockSpec((1, ct), lambda j: (0, j)),
            pl.BlockSpec((1, ct), lambda j: (0, j)),
        ],
        out_specs=pl.BlockSpec((n, ct), lambda j: (0, j)),
        out_shape=jax.ShapeDtypeStruct((n, oc), jnp.float32),
    )(h, gamma.reshape(1, oc), beta.reshape(1, oc))


def _pool_max(h, cluster):
    n, oc = h.shape
    ct = 128

    def body(h_ref, cl_ref, out_ref):
        hb = h_ref[...]
        cl = cl_ref[...]
        rows = [jnp.max(jnp.where(cl == c, hb, -jnp.inf), axis=0,
                        keepdims=True) for c in range(4)]
        out_ref[...] = jnp.concatenate(rows + rows, axis=0)

    return pl.pallas_call(
        body,
        grid=(oc // ct,),
        in_specs=[
            pl.BlockSpec((n, ct), lambda j: (0, j)),
            pl.BlockSpec((n, 1), lambda j: (0, 0)),
        ],
        out_specs=pl.BlockSpec((8, ct), lambda j: (0, j)),
        out_shape=jax.ShapeDtypeStruct((8, oc), jnp.float32),
    )(h, cluster.reshape(n, 1))


def _head(feat, lin1, lin2):
    def body(feat_ref, l1_ref, l2_ref, out_ref):
        t = jnp.dot(feat_ref[...], l1_ref[...],
                    preferred_element_type=jnp.float32)
        out_ref[...] = jnp.dot(t, l2_ref[...],
                               preferred_element_type=jnp.float32)

    return pl.pallas_call(
        body,
        out_shape=jax.ShapeDtypeStruct((1, lin2.shape[1]), jnp.float32),
    )(feat, lin1, lin2)


def kernel(x, edge_attr, W1, root1, bias1, g1, be1, W2, root2, bias2, g2, be2,
           W3, root3, bias3, g3, be3, W4, root4, bias4, g4, be4,
           Wm, rootm, biasm, gm, bem, lin1, lin2, edge_index, pos):
    src = edge_index[0]
    dst = edge_index[1]
    n = x.shape[0]
    n_edges = src.shape[0]
    G = n // TN

    dst_s, src_s, ea0, ea1, ea2 = lax.sort(
        (dst.astype(jnp.int32), src.astype(jnp.int32),
         edge_attr[:, 0], edge_attr[:, 1], edge_attr[:, 2]),
        dimension=0, is_stable=False, num_keys=1)
    w8, k8 = _basis(jnp.stack([ea0, ea1, ea2], axis=1))
    w8f = w8.reshape(-1)
    r8f = ((dst_s % TN)[:, None] * 64 + k8).reshape(-1).astype(jnp.int32)
    starts = jnp.searchsorted(dst_s, jnp.arange(G + 1, dtype=jnp.int32) * TN
                              ).astype(jnp.int32)
    gp = ((G + 16 + 7) // 8) * 8
    starts_p = jnp.concatenate(
        [starts, jnp.full((gp - G - 1,), n_edges, jnp.int32)])

    node_b = jnp.searchsorted(dst_s, jnp.arange(n + 1, dtype=jnp.int32))
    deg = (node_b[1:] - node_b[:-1]).astype(jnp.float32).reshape(n, 1)
    invdeg = 1.0 / jnp.maximum(deg, 1.0)

    x16 = jnp.concatenate([x, jnp.zeros((n, 15), jnp.float32)], axis=1)
    a1p = _sc_layer(x16, src_s, w8f, r8f, starts_p, 16)
    w1p = jnp.pad(W1, ((0, 0), (0, 15), (0, 0)))
    h = _mm_layer(a1p, w1p, x, root1, bias1, invdeg)
    h = _bn(h, g1, be1)
    for (W, root, bias, gm_, be_) in ((W2, root2, bias2, g2, be2),
                                      (W3, root3, bias3, g3, be3),
                                      (W4, root4, bias4, g4, be4)):
        a = _sc_layer(h, src_s, w8f, r8f, starts_p, h.shape[1])
        h = _mm_layer(a, W, h, root, bias, invdeg)
        h = _bn(h, gm_, be_)

    cx = jnp.clip(jnp.floor(pos[:, 0] / 120.0), 0, 1).astype(jnp.int32)
    cy = jnp.clip(jnp.floor(pos[:, 1] / 90.0), 0, 1).astype(jnp.int32)
    cluster = cy * 2 + cx
    pooled = _pool_max(h, cluster)
    feat = pooled[:4].reshape(1, 4 * 512)
    labels = _head(feat, lin1, lin2)

    xm = jnp.zeros((n, 1), jnp.float32)
    return labels, xm


# speedup vs baseline: 3.8804x; 1.0070x over previous
import functools

import jax
import jax.numpy as jnp
from jax import lax
from jax.experimental import pallas as pl
from jax.experimental.pallas import tpu as pltpu
from jax.experimental.pallas import tpu_sc as plsc

KS = 4
K3 = KS ** 3
TN = 4
C = 64
NW = 32


def _basis(edge_attr):
    u = jnp.clip(edge_attr, 0.0, 1.0) * (KS - 1)
    base = jnp.floor(u)
    frac = u - base
    base_i = base.astype(jnp.int32)
    ws, kidxs = [], []
    for o0 in (0, 1):
        for o1 in (0, 1):
            for o2 in (0, 1):
                o = (o0, o1, o2)
                w = jnp.ones(edge_attr.shape[:1], jnp.float32)
                kidx = jnp.zeros(edge_attr.shape[:1], jnp.int32)
                mult = 1
                for d in range(3):
                    fd = frac[:, d]
                    w = w * (fd if o[d] == 1 else (1.0 - fd))
                    kidx = kidx + jnp.clip(base_i[:, d] + o[d], 0, KS - 1) * mult
                    mult = mult * KS
                ws.append(w)
                kidxs.append(kidx)
    return jnp.stack(ws, axis=1), jnp.stack(kidxs, axis=1)


def _sc_layer(x, src_s, dst_s, ea0, ea1, ea2, starts_p, ic, want_deg=False):
    n_nodes = x.shape[0]
    G = n_nodes // TN
    GP = starts_p.shape[0]
    J = ic // 16
    ROWS = TN * 64
    mesh = plsc.VectorSubcoreMesh(core_axis_name="c", subcore_axis_name="s")
    out_type = [jax.ShapeDtypeStruct((G * ROWS * ic,), jnp.float32)]
    if want_deg:
        out_type.append(jax.ShapeDtypeStruct((G * 16,), jnp.float32))
    scratch = [
        pltpu.VMEM((GP,), jnp.int32),
        pltpu.VMEM((C,), jnp.int32),
        pltpu.VMEM((C + 16,), jnp.int32),
        pltpu.VMEM((C + 16,), jnp.float32),
        pltpu.VMEM((C + 16,), jnp.float32),
        pltpu.VMEM((C + 16,), jnp.float32),
        pltpu.VMEM((C, ic), jnp.float32),
        pltpu.VMEM((ROWS * ic,), jnp.float32),
        pltpu.SemaphoreType.DMA,
    ]
    if want_deg:
        scratch.append(pltpu.VMEM((16,), jnp.float32))

    @functools.partial(
        pl.kernel,
        out_type=tuple(out_type),
        mesh=mesh,
        scratch_types=tuple(scratch),
        compiler_params=pltpu.CompilerParams(use_tc_tiling_on_sc=False),
    )
    def k(x_hbm, src_hbm, dst_hbm, ea0_hbm, ea1_hbm, ea2_hbm, starts_hbm,
          *rest):
        if want_deg:
            (a_hbm, deg_hbm, starts_v, src_v, dst_v, e0_v, e1_v, e2_v,
             rows_v, a_v, sem, deg_v) = rest
        else:
            (a_hbm, starts_v, src_v, dst_v, e0_v, e1_v, e2_v,
             rows_v, a_v, sem) = rest
        wid = lax.axis_index("s") * 2 + lax.axis_index("c")
        pltpu.sync_copy(starts_hbm, starts_v)
        iota = lax.iota(jnp.int32, 16)
        ntw = (jnp.int32(G) - wid + (NW - 1)) // NW

        def tile_body(i, _):
            t = wid + i * NW

            def zbody(z, _):
                for u in range(8):
                    a_v[pl.ds((z * 8 + u) * 16, 16)] = jnp.zeros((16,),
                                                                 jnp.float32)
                return 0

            lax.fori_loop(0, ROWS * J // 8, zbody, 0)
            if want_deg:
                deg_v[...] = jnp.zeros((16,), jnp.float32)
            sv = starts_v[pl.ds(t, 16)]
            s = sv[0]
            e = sv[1]
            cb0 = s // C
            cb1 = (e + (C - 1)) // C

            def chunk_body(cb, _):
                pltpu.sync_copy(src_hbm.at[pl.ds(cb * C, C)], src_v)
                pltpu.sync_copy(dst_hbm.at[pl.ds(cb * C, C)],
                                dst_v.at[pl.ds(0, C)])
                pltpu.sync_copy(ea0_hbm.at[pl.ds(cb * C, C)],
                                e0_v.at[pl.ds(0, C)])
                pltpu.sync_copy(ea1_hbm.at[pl.ds(cb * C, C)],
                                e1_v.at[pl.ds(0, C)])
                pltpu.sync_copy(ea2_hbm.at[pl.ds(cb * C, C)],
                                e2_v.at[pl.ds(0, C)])
                pltpu.async_copy(x_hbm.at[src_v], rows_v, sem).wait()
                elo = jnp.maximum(s - cb * C, 0)
                ehi = jnp.minimum(e - cb * C, C)

                def edge_body(el, _):
                    segs = [rows_v[el, pl.ds(16 * j, 16)] for j in range(J)]
                    d = dst_v[pl.ds(el, 16)][0]
                    e0 = e0_v[pl.ds(el, 16)][0]
                    e1 = e1_v[pl.ds(el, 16)][0]
                    e2 = e2_v[pl.ds(el, 16)][0]
                    ld = d % TN
                    u0 = jnp.minimum(jnp.maximum(e0, 0.0), 1.0) * 3.0
                    u1 = jnp.minimum(jnp.maximum(e1, 0.0), 1.0) * 3.0
                    u2 = jnp.minimum(jnp.maximum(e2, 0.0), 1.0) * 3.0
                    def flr(u):
                        return ((u >= 1.0).astype(jnp.int32)
                                + (u >= 2.0).astype(jnp.int32)
                                + (u >= 3.0).astype(jnp.int32))

                    b0 = flr(u0)
                    b1 = flr(u1)
                    b2 = flr(u2)
                    f0 = u0 - b0.astype(jnp.float32)
                    f1 = u1 - b1.astype(jnp.float32)
                    f2 = u2 - b2.astype(jnp.float32)
                    g0 = 1.0 - f0
                    g1 = 1.0 - f1
                    g2 = 1.0 - f2
                    k0 = (b0, jnp.minimum(b0 + 1, 3))
                    k1 = (b1 * 4, jnp.minimum(b1 + 1, 3) * 4)
                    k2 = (b2 * 16, jnp.minimum(b2 + 1, 3) * 16)
                    p = ((g0 * g1, g0 * f1), (f0 * g1, f0 * f1))
                    node = ld * (64 * ic)
                    for o0 in (0, 1):
                        for o1 in (0, 1):
                            w01 = p[o0][o1]
                            k01 = k0[o0] + k1[o1]
                            for o2 in (0, 1):
                                w = w01 * (f2 if o2 else g2)
                                base = node + (k01 + k2[o2]) * ic
                                for j in range(J):
                                    plsc.addupdate(
                                        a_v.at[pl.ds(base + 16 * j, 16)],
                                        w * segs[j])
                    if want_deg:
                        plsc.addupdate(
                            deg_v.at[pl.ds(0, 16)],
                            jnp.where(iota == ld, 1.0, 0.0).astype(
                                jnp.float32))
                    return 0

                lax.fori_loop(elo, ehi, edge_body, 0)
                return 0

            lax.fori_loop(cb0, cb1, chunk_body, 0)
            pltpu.sync_copy(a_v, a_hbm.at[pl.ds(t * ROWS * ic, ROWS * ic)])
            if want_deg:
                pltpu.sync_copy(deg_v, deg_hbm.at[pl.ds(t * 16, 16)])
            return 0

        lax.fori_loop(0, ntw, tile_body, 0)

    res = k(x, src_s, dst_s, ea0, ea1, ea2, starts_p)
    return res if want_deg else res[0]


def _mm_layer(a_flat, W, x, root, bias, invdeg):
    n, ic = x.shape
    kd = W.shape[0] * W.shape[1]
    oc = W.shape[2]
    a2 = a_flat.reshape(n, kd)
    wr = W.reshape(kd, oc)
    kt = min(kd, 2048)
    nk = kd // kt
    nt = 1000
    ni = n // nt

    def body(a_ref, w_ref, x_ref, root_ref, bias_ref, invdeg_ref, out_ref,
             acc_ref):
        kk = pl.program_id(1)

        @pl.when(kk == 0)
        def _():
            acc_ref[...] = jnp.zeros_like(acc_ref)

        acc_ref[...] += jnp.dot(a_ref[...], w_ref[...],
                                preferred_element_type=jnp.float32)

        @pl.when(kk == nk - 1)
        def _():
            h = (acc_ref[...] * invdeg_ref[...]
                 + jnp.dot(x_ref[...], root_ref[...],
                           preferred_element_type=jnp.float32)
                 + bias_ref[...])
            out_ref[...] = jnp.where(h > 0, h, jnp.exp(h) - 1.0)

    return pl.pallas_call(
        body,
        grid=(ni, nk),
        in_specs=[
            pl.BlockSpec((nt, kt), lambda i, k: (i, k)),
            pl.BlockSpec((kt, oc), lambda i, k: (k, 0)),
            pl.BlockSpec((nt, ic), lambda i, k: (i, 0)),
            pl.BlockSpec((ic, oc), lambda i, k: (0, 0)),
            pl.BlockSpec((1, oc), lambda i, k: (0, 0)),
            pl.BlockSpec((nt, 1), lambda i, k: (i, 0)),
        ],
        out_specs=pl.BlockSpec((nt, oc), lambda i, k: (i, 0)),
        out_shape=jax.ShapeDtypeStruct((n, oc), jnp.float32),
        scratch_shapes=[pltpu.VMEM((nt, oc), jnp.float32)],
    )(a2, wr, x, root.reshape(ic, oc), bias.reshape(1, oc), invdeg)


def _bn(h, gamma, beta):
    n, oc = h.shape
    ct = min(oc, 128)

    def body(h_ref, g_ref, b_ref, out_ref):
        hb = h_ref[...]
        m = jnp.mean(hb, axis=0, keepdims=True)
        v = jnp.mean((hb - m) ** 2, axis=0, keepdims=True)
        out_ref[...] = g_ref[...] * (hb - m) / jnp.sqrt(v + 1e-5) + b_ref[...]

    return pl.pallas_call(
        body,
        grid=(oc // ct,),
        in_specs=[
            pl.BlockSpec((n, ct), lambda j: (0, j)),
            pl.BlockSpec((1, ct), lambda j: (0, j)),
            pl.BlockSpec((1, ct), lambda j: (0, j)),
        ],
        out_specs=pl.BlockSpec((n, ct), lambda j: (0, j)),
        out_shape=jax.ShapeDtypeStruct((n, oc), jnp.float32),
    )(h, gamma.reshape(1, oc), beta.reshape(1, oc))


def _pool_max(h, cluster):
    n, oc = h.shape
    ct = 128

    def body(h_ref, cl_ref, out_ref):
        hb = h_ref[...]
        cl = cl_ref[...]
        rows = [jnp.max(jnp.where(cl == c, hb, -jnp.inf), axis=0,
                        keepdims=True) for c in range(4)]
        out_ref[...] = jnp.concatenate(rows + rows, axis=0)

    return pl.pallas_call(
        body,
        grid=(oc // ct,),
        in_specs=[
            pl.BlockSpec((n, ct), lambda j: (0, j)),
            pl.BlockSpec((n, 1), lambda j: (0, 0)),
        ],
        out_specs=pl.BlockSpec((8, ct), lambda j: (0, j)),
        out_shape=jax.ShapeDtypeStruct((8, oc), jnp.float32),
    )(h, cluster.reshape(n, 1))


def _head(feat, lin1, lin2):
    def body(feat_ref, l1_ref, l2_ref, out_ref):
        t = jnp.dot(feat_ref[...], l1_ref[...],
                    preferred_element_type=jnp.float32)
        out_ref[...] = jnp.dot(t, l2_ref[...],
                               preferred_element_type=jnp.float32)

    return pl.pallas_call(
        body,
        out_shape=jax.ShapeDtypeStruct((1, lin2.shape[1]), jnp.float32),
    )(feat, lin1, lin2)


def kernel(x, edge_attr, W1, root1, bias1, g1, be1, W2, root2, bias2, g2, be2,
           W3, root3, bias3, g3, be3, W4, root4, bias4, g4, be4,
           Wm, rootm, biasm, gm, bem, lin1, lin2, edge_index, pos):
    src = edge_index[0]
    dst = edge_index[1]
    n = x.shape[0]
    n_edges = src.shape[0]
    G = n // TN

    dst_s, src_s, ea0, ea1, ea2 = lax.sort(
        (dst.astype(jnp.int32), src.astype(jnp.int32),
         edge_attr[:, 0], edge_attr[:, 1], edge_attr[:, 2]),
        dimension=0, is_stable=False, num_keys=1)
    starts = jnp.searchsorted(dst_s, jnp.arange(G + 1, dtype=jnp.int32) * TN
                              ).astype(jnp.int32)
    gp = ((G + 16 + 7) // 8) * 8
    starts_p = jnp.concatenate(
        [starts, jnp.full((gp - G - 1,), n_edges, jnp.int32)])

    x16 = jnp.concatenate([x, jnp.zeros((n, 15), jnp.float32)], axis=1)
    a1p, degp = _sc_layer(x16, src_s, dst_s, ea0, ea1, ea2, starts_p, 16,
                          want_deg=True)
    deg = degp.reshape(G, 16)[:, :TN].reshape(n, 1)
    invdeg = 1.0 / jnp.maximum(deg, 1.0)
    w1p = jnp.pad(W1, ((0, 0), (0, 15), (0, 0)))
    h = _mm_layer(a1p, w1p, x, root1, bias1, invdeg)
    h = _bn(h, g1, be1)
    for (W, root, bias, gm_, be_) in ((W2, root2, bias2, g2, be2),
                                      (W3, root3, bias3, g3, be3),
                                      (W4, root4, bias4, g4, be4)):
        a = _sc_layer(h, src_s, dst_s, ea0, ea1, ea2, starts_p, h.shape[1])
        h = _mm_layer(a, W, h, root, bias, invdeg)
        h = _bn(h, gm_, be_)

    cx = jnp.clip(jnp.floor(pos[:, 0] / 120.0), 0, 1).astype(jnp.int32)
    cy = jnp.clip(jnp.floor(pos[:, 1] / 90.0), 0, 1).astype(jnp.int32)
    cluster = cy * 2 + cx
    pooled = _pool_max(h, cluster)
    feat = pooled[:4].reshape(1, 4 * 512)
    labels = _head(feat, lin1, lin2)

    xm = jnp.zeros((n, 1), jnp.float32)
    return labels, xm


# speedup vs baseline: 4.8410x; 1.2476x over previous
import functools

import jax
import jax.numpy as jnp
from jax import lax
from jax.experimental import pallas as pl
from jax.experimental.pallas import tpu as pltpu
from jax.experimental.pallas import tpu_sc as plsc

KS = 4
K3 = KS ** 3
TN = 4
C = 64
NW = 32


def _basis(edge_attr):
    u = jnp.clip(edge_attr, 0.0, 1.0) * (KS - 1)
    base = jnp.floor(u)
    frac = u - base
    base_i = base.astype(jnp.int32)
    ws, kidxs = [], []
    for o0 in (0, 1):
        for o1 in (0, 1):
            for o2 in (0, 1):
                o = (o0, o1, o2)
                w = jnp.ones(edge_attr.shape[:1], jnp.float32)
                kidx = jnp.zeros(edge_attr.shape[:1], jnp.int32)
                mult = 1
                for d in range(3):
                    fd = frac[:, d]
                    w = w * (fd if o[d] == 1 else (1.0 - fd))
                    kidx = kidx + jnp.clip(base_i[:, d] + o[d], 0, KS - 1) * mult
                    mult = mult * KS
                ws.append(w)
                kidxs.append(kidx)
    return jnp.stack(ws, axis=1), jnp.stack(kidxs, axis=1)


def _sc_layer(x, src_s, w8f, r8f, dst_s, starts_p, ic, want_deg=False):
    n_nodes = x.shape[0]
    G = n_nodes // TN
    GP = starts_p.shape[0]
    J = ic // 16
    ROWS = TN * 64
    tiled = ic >= 128
    CE = 128 if tiled else 64
    mesh = plsc.VectorSubcoreMesh(core_axis_name="c", subcore_axis_name="s")
    if tiled:
        out_type = [jax.ShapeDtypeStruct((G * ROWS, ic), jnp.float32)]
        a_scratch = pltpu.VMEM((ROWS, ic), jnp.float32)
        params = None
    else:
        out_type = [jax.ShapeDtypeStruct((G * ROWS * ic,), jnp.float32)]
        a_scratch = pltpu.VMEM((ROWS * ic,), jnp.float32)
        params = pltpu.CompilerParams(use_tc_tiling_on_sc=False)
    if want_deg:
        out_type.append(jax.ShapeDtypeStruct((G * 16,), jnp.float32))
    scratch = [
        pltpu.VMEM((GP,), jnp.int32),
        pltpu.VMEM((CE,), jnp.int32),
        pltpu.VMEM((CE * 8 + 16,), jnp.float32),
        pltpu.VMEM((CE * 8 + 16,), jnp.int32),
        pltpu.VMEM((CE + 16,), jnp.int32),
        pltpu.VMEM((CE, ic), jnp.float32),
        a_scratch,
        pltpu.SemaphoreType.DMA,
    ]
    if want_deg:
        scratch.append(pltpu.VMEM((16,), jnp.float32))

    @functools.partial(
        pl.kernel,
        out_type=tuple(out_type),
        mesh=mesh,
        scratch_types=tuple(scratch),
        compiler_params=params,
    )
    def k(x_hbm, src_hbm, w_hbm, r_hbm, dst_hbm, starts_hbm, *rest):
        if want_deg:
            (a_hbm, deg_hbm, starts_v, src_v, w_v, r_v, dst_v,
             rows_v, a_v, sem, deg_v) = rest
        else:
            (a_hbm, starts_v, src_v, w_v, r_v, dst_v,
             rows_v, a_v, sem) = rest
        wid = lax.axis_index("s") * 2 + lax.axis_index("c")
        pltpu.sync_copy(starts_hbm, starts_v)
        iota = lax.iota(jnp.int32, 16)
        ntw = (jnp.int32(G) - wid + (NW - 1)) // NW

        def tile_body(i, _):
            t = wid + i * NW

            if tiled:
                def zbody(z, _):
                    for j in range(J):
                        a_v[z, pl.ds(16 * j, 16)] = jnp.zeros((16,),
                                                              jnp.float32)
                    return 0

                lax.fori_loop(0, ROWS, zbody, 0)
            else:
                def zbody(z, _):
                    for u in range(8):
                        a_v[pl.ds((z * 8 + u) * 16, 16)] = jnp.zeros(
                            (16,), jnp.float32)
                    return 0

                lax.fori_loop(0, ROWS * J // 8, zbody, 0)
            if want_deg:
                deg_v[...] = jnp.zeros((16,), jnp.float32)
            sv = starts_v[pl.ds(t, 16)]
            s = sv[0]
            e = sv[1]
            cb0 = s // CE
            cb1 = (e + (CE - 1)) // CE

            def chunk_body(cb, _):
                pltpu.sync_copy(src_hbm.at[pl.ds(cb * CE, CE)], src_v)
                pltpu.sync_copy(w_hbm.at[pl.ds(cb * CE * 8, CE * 8)],
                                w_v.at[pl.ds(0, CE * 8)])
                pltpu.sync_copy(r_hbm.at[pl.ds(cb * CE * 8, CE * 8)],
                                r_v.at[pl.ds(0, CE * 8)])
                if want_deg:
                    pltpu.sync_copy(dst_hbm.at[pl.ds(cb * CE, CE)],
                                    dst_v.at[pl.ds(0, CE)])
                pltpu.async_copy(x_hbm.at[src_v], rows_v, sem).wait()
                elo = jnp.maximum(s - cb * CE, 0)
                ehi = jnp.minimum(e - cb * CE, CE)

                def edge_body(el, _):
                    segs = [rows_v[el, pl.ds(16 * j, 16)] for j in range(J)]
                    wv = w_v[pl.ds(el * 8, 16)]
                    rv = r_v[pl.ds(el * 8, 16)]
                    for c in range(8):
                        w = wv[c]
                        r = rv[c]
                        if tiled:
                            for j in range(J):
                                plsc.addupdate(
                                    a_v.at[r, pl.ds(16 * j, 16)],
                                    w * segs[j])
                        else:
                            base = r * ic
                            for j in range(J):
                                plsc.addupdate(
                                    a_v.at[pl.ds(base + 16 * j, 16)],
                                    w * segs[j])
                    if want_deg:
                        d = dst_v[pl.ds(el, 16)][0]
                        plsc.addupdate(
                            deg_v.at[pl.ds(0, 16)],
                            jnp.where(iota == d % TN, 1.0, 0.0).astype(
                                jnp.float32))
                    return 0

                lax.fori_loop(elo, ehi, edge_body, 0)
                return 0

            lax.fori_loop(cb0, cb1, chunk_body, 0)
            if tiled:
                pltpu.sync_copy(a_v, a_hbm.at[pl.ds(t * ROWS, ROWS), :])
            else:
                pltpu.sync_copy(a_v,
                                a_hbm.at[pl.ds(t * ROWS * ic, ROWS * ic)])
            if want_deg:
                pltpu.sync_copy(deg_v, deg_hbm.at[pl.ds(t * 16, 16)])
            return 0

        lax.fori_loop(0, ntw, tile_body, 0)

    res = k(x, src_s, w8f, r8f, dst_s, starts_p)
    return res if want_deg else res[0]


def _mm_layer(a_flat, W, x, root, bias, invdeg):
    n, ic = x.shape
    kd = W.shape[0] * W.shape[1]
    oc = W.shape[2]
    a2 = a_flat.reshape(n, kd)
    wr = W.reshape(kd, oc)
    kt = min(kd, 2048)
    nk = kd // kt
    nt = 1000
    ni = n // nt

    def body(a_ref, w_ref, x_ref, root_ref, bias_ref, invdeg_ref, out_ref,
             acc_ref):
        kk = pl.program_id(1)

        @pl.when(kk == 0)
        def _():
            acc_ref[...] = jnp.zeros_like(acc_ref)

        acc_ref[...] += jnp.dot(a_ref[...], w_ref[...],
                                preferred_element_type=jnp.float32)

        @pl.when(kk == nk - 1)
        def _():
            h = (acc_ref[...] * invdeg_ref[...]
                 + jnp.dot(x_ref[...], root_ref[...],
                           preferred_element_type=jnp.float32)
                 + bias_ref[...])
            out_ref[...] = jnp.where(h > 0, h, jnp.exp(h) - 1.0)

    return pl.pallas_call(
        body,
        grid=(ni, nk),
        in_specs=[
            pl.BlockSpec((nt, kt), lambda i, k: (i, k)),
            pl.BlockSpec((kt, oc), lambda i, k: (k, 0)),
            pl.BlockSpec((nt, ic), lambda i, k: (i, 0)),
            pl.BlockSpec((ic, oc), lambda i, k: (0, 0)),
            pl.BlockSpec((1, oc), lambda i, k: (0, 0)),
            pl.BlockSpec((nt, 1), lambda i, k: (i, 0)),
        ],
        out_specs=pl.BlockSpec((nt, oc), lambda i, k: (i, 0)),
        out_shape=jax.ShapeDtypeStruct((n, oc), jnp.float32),
        scratch_shapes=[pltpu.VMEM((nt, oc), jnp.float32)],
    )(a2, wr, x, root.reshape(ic, oc), bias.reshape(1, oc), invdeg)


def _mm_layer_t(a2d, W, x, root, bias, invdeg):
    n, ic = x.shape
    oc = W.shape[2]
    a3 = a2d.reshape(n, 64, ic)
    kb = 8
    nk = 64 // kb
    nt = 1000
    ni = n // nt

    def body(a_ref, w_ref, x_ref, root_ref, bias_ref, invdeg_ref, out_ref,
             acc_ref):
        kk = pl.program_id(1)

        @pl.when(kk == 0)
        def _():
            acc_ref[...] = jnp.zeros_like(acc_ref)

        for u in range(kb):
            acc_ref[...] += jnp.dot(a_ref[:, u, :], w_ref[u],
                                    preferred_element_type=jnp.float32)

        @pl.when(kk == nk - 1)
        def _():
            h = (acc_ref[...] * invdeg_ref[...]
                 + jnp.dot(x_ref[...], root_ref[...],
                           preferred_element_type=jnp.float32)
                 + bias_ref[...])
            out_ref[...] = jnp.where(h > 0, h, jnp.exp(h) - 1.0)

    return pl.pallas_call(
        body,
        grid=(ni, nk),
        in_specs=[
            pl.BlockSpec((nt, kb, ic), lambda i, k: (i, k, 0)),
            pl.BlockSpec((kb, ic, oc), lambda i, k: (k, 0, 0)),
            pl.BlockSpec((nt, ic), lambda i, k: (i, 0)),
            pl.BlockSpec((ic, oc), lambda i, k: (0, 0)),
            pl.BlockSpec((1, oc), lambda i, k: (0, 0)),
            pl.BlockSpec((nt, 1), lambda i, k: (i, 0)),
        ],
        out_specs=pl.BlockSpec((nt, oc), lambda i, k: (i, 0)),
        out_shape=jax.ShapeDtypeStruct((n, oc), jnp.float32),
        scratch_shapes=[pltpu.VMEM((nt, oc), jnp.float32)],
    )(a3, W, x, root.reshape(ic, oc), bias.reshape(1, oc), invdeg)


def _bn(h, gamma, beta):
    n, oc = h.shape
    ct = min(oc, 128)

    def body(h_ref, g_ref, b_ref, out_ref):
        hb = h_ref[...]
        m = jnp.mean(hb, axis=0, keepdims=True)
        v = jnp.mean((hb - m) ** 2, axis=0, keepdims=True)
        out_ref[...] = g_ref[...] * (hb - m) / jnp.sqrt(v + 1e-5) + b_ref[...]

    return pl.pallas_call(
        body,
        grid=(oc // ct,),
        in_specs=[
            pl.BlockSpec((n, ct), lambda j: (0, j)),
            pl.BlockSpec((1, ct), lambda j: (0, j)),
            pl.BlockSpec((1, ct), lambda j: (0, j)),
        ],
        out_specs=pl.BlockSpec((n, ct), lambda j: (0, j)),
        out_shape=jax.ShapeDtypeStruct((n, oc), jnp.float32),
    )(h, gamma.reshape(1, oc), beta.reshape(1, oc))


def _pool_max(h, cluster):
    n, oc = h.shape
    ct = 128

    def body(h_ref, cl_ref, out_ref):
        hb = h_ref[...]
        cl = cl_ref[...]
        rows = [jnp.max(jnp.where(cl == c, hb, -jnp.inf), axis=0,
                        keepdims=True) for c in range(4)]
        out_ref[...] = jnp.concatenate(rows + rows, axis=0)

    return pl.pallas_call(
        body,
        grid=(oc // ct,),
        in_specs=[
            pl.BlockSpec((n, ct), lambda j: (0, j)),
            pl.BlockSpec((n, 1), lambda j: (0, 0)),
        ],
        out_specs=pl.BlockSpec((8, ct), lambda j: (0, j)),
        out_shape=jax.ShapeDtypeStruct((8, oc), jnp.float32),
    )(h, cluster.reshape(n, 1))


def _head(feat, lin1, lin2):
    def body(feat_ref, l1_ref, l2_ref, out_ref):
        t = jnp.dot(feat_ref[...], l1_ref[...],
                    preferred_element_type=jnp.float32)
        out_ref[...] = jnp.dot(t, l2_ref[...],
                               preferred_element_type=jnp.float32)

    return pl.pallas_call(
        body,
        out_shape=jax.ShapeDtypeStruct((1, lin2.shape[1]), jnp.float32),
    )(feat, lin1, lin2)


def kernel(x, edge_attr, W1, root1, bias1, g1, be1, W2, root2, bias2, g2, be2,
           W3, root3, bias3, g3, be3, W4, root4, bias4, g4, be4,
           Wm, rootm, biasm, gm, bem, lin1, lin2, edge_index, pos):
    src = edge_index[0]
    dst = edge_index[1]
    n = x.shape[0]
    n_edges = src.shape[0]
    G = n // TN

    dst_s, src_s, ea0, ea1, ea2 = lax.sort(
        (dst.astype(jnp.int32), src.astype(jnp.int32),
         edge_attr[:, 0], edge_attr[:, 1], edge_attr[:, 2]),
        dimension=0, is_stable=False, num_keys=1)
    w8, k8 = _basis(jnp.stack([ea0, ea1, ea2], axis=1))
    w8f = w8.reshape(-1)
    r8f = ((dst_s % TN)[:, None] * 64 + k8).reshape(-1).astype(jnp.int32)
    starts = jnp.searchsorted(dst_s, jnp.arange(G + 1, dtype=jnp.int32) * TN
                              ).astype(jnp.int32)
    gp = ((G + 16 + 7) // 8) * 8
    starts_p = jnp.concatenate(
        [starts, jnp.full((gp - G - 1,), n_edges, jnp.int32)])

    x16 = jnp.concatenate([x, jnp.zeros((n, 15), jnp.float32)], axis=1)
    a1p, degp = _sc_layer(x16, src_s, w8f, r8f, dst_s, starts_p, 16,
                          want_deg=True)
    deg = degp.reshape(G, 16)[:, :TN].reshape(n, 1)
    invdeg = 1.0 / jnp.maximum(deg, 1.0)
    w1p = jnp.pad(W1, ((0, 0), (0, 15), (0, 0)))
    h = _mm_layer(a1p, w1p, x, root1, bias1, invdeg)
    h = _bn(h, g1, be1)
    for (W, root, bias, gm_, be_) in ((W2, root2, bias2, g2, be2),
                                      (W3, root3, bias3, g3, be3),
                                      (W4, root4, bias4, g4, be4)):
        a = _sc_layer(h, src_s, w8f, r8f, dst_s, starts_p, h.shape[1])
        if h.shape[1] >= 128:
            h = _mm_layer_t(a, W, h, root, bias, invdeg)
        else:
            h = _mm_layer(a, W, h, root, bias, invdeg)
        h = _bn(h, gm_, be_)

    cx = jnp.clip(jnp.floor(pos[:, 0] / 120.0), 0, 1).astype(jnp.int32)
    cy = jnp.clip(jnp.floor(pos[:, 1] / 90.0), 0, 1).astype(jnp.int32)
    cluster = cy * 2 + cx
    pooled = _pool_max(h, cluster)
    feat = pooled[:4].reshape(1, 4 * 512)
    labels = _head(feat, lin1, lin2)

    xm = jnp.zeros((n, 1), jnp.float32)
    return labels, xm


# speedup vs baseline: 5.1302x; 1.0597x over previous
import functools

import jax
import jax.numpy as jnp
from jax import lax
from jax.experimental import pallas as pl
from jax.experimental.pallas import tpu as pltpu
from jax.experimental.pallas import tpu_sc as plsc

KS = 4
K3 = KS ** 3
TN = 4
C = 64
NW = 32


def _basis(edge_attr):
    u = jnp.clip(edge_attr, 0.0, 1.0) * (KS - 1)
    base = jnp.floor(u)
    frac = u - base
    base_i = base.astype(jnp.int32)
    ws, kidxs = [], []
    for o0 in (0, 1):
        for o1 in (0, 1):
            for o2 in (0, 1):
                o = (o0, o1, o2)
                w = jnp.ones(edge_attr.shape[:1], jnp.float32)
                kidx = jnp.zeros(edge_attr.shape[:1], jnp.int32)
                mult = 1
                for d in range(3):
                    fd = frac[:, d]
                    w = w * (fd if o[d] == 1 else (1.0 - fd))
                    kidx = kidx + jnp.clip(base_i[:, d] + o[d], 0, KS - 1) * mult
                    mult = mult * KS
                ws.append(w)
                kidxs.append(kidx)
    return jnp.stack(ws, axis=1), jnp.stack(kidxs, axis=1)


def _sc_layer(x, src_s, w8f, r8f, dst_s, starts_p, ic, want_deg=False):
    n_nodes = x.shape[0]
    G = n_nodes // TN
    GP = starts_p.shape[0]
    J = ic // 16
    ROWS = TN * 64
    tiled = ic >= 128
    CE = 128 if tiled else 64
    mesh = plsc.VectorSubcoreMesh(core_axis_name="c", subcore_axis_name="s")
    if tiled:
        out_type = [jax.ShapeDtypeStruct((G * ROWS, ic), jnp.float32)]
        a_scratch = pltpu.VMEM((ROWS, ic), jnp.float32)
        params = None
    else:
        out_type = [jax.ShapeDtypeStruct((G * ROWS * ic,), jnp.float32)]
        a_scratch = pltpu.VMEM((ROWS * ic,), jnp.float32)
        params = pltpu.CompilerParams(use_tc_tiling_on_sc=False)
    if want_deg:
        out_type.append(jax.ShapeDtypeStruct((G * 16,), jnp.float32))
    scratch = [
        pltpu.VMEM((GP,), jnp.int32),
        pltpu.VMEM((CE,), jnp.int32),
        pltpu.VMEM((CE * 8 + 16,), jnp.float32),
        pltpu.VMEM((CE * 8 + 16,), jnp.int32),
        pltpu.VMEM((CE + 16,), jnp.int32),
        pltpu.VMEM((CE, ic), jnp.float32),
        a_scratch,
        pltpu.SemaphoreType.DMA,
        pltpu.SemaphoreType.DMA,
    ]
    if want_deg:
        scratch.append(pltpu.VMEM((16,), jnp.float32))

    @functools.partial(
        pl.kernel,
        out_type=tuple(out_type),
        mesh=mesh,
        scratch_types=tuple(scratch),
        compiler_params=params,
    )
    def k(x_hbm, src_hbm, w_hbm, r_hbm, dst_hbm, starts_hbm, *rest):
        if want_deg:
            (a_hbm, deg_hbm, starts_v, src_v, w_v, r_v, dst_v,
             rows_v, a_v, sem, osem, deg_v) = rest
        else:
            (a_hbm, starts_v, src_v, w_v, r_v, dst_v,
             rows_v, a_v, sem, osem) = rest
        wid = lax.axis_index("s") * 2 + lax.axis_index("c")
        pltpu.sync_copy(starts_hbm, starts_v)
        iota = lax.iota(jnp.int32, 16)
        ntw = (jnp.int32(G) - wid + (NW - 1)) // NW

        def _out_view(t):
            if tiled:
                return a_hbm.at[pl.ds(t * ROWS, ROWS), :]
            return a_hbm.at[pl.ds(t * ROWS * ic, ROWS * ic)]

        def meta_and_gather(cb):
            pltpu.sync_copy(src_hbm.at[pl.ds(cb * CE, CE)], src_v)
            pltpu.sync_copy(w_hbm.at[pl.ds(cb * CE * 8, CE * 8)],
                            w_v.at[pl.ds(0, CE * 8)])
            pltpu.sync_copy(r_hbm.at[pl.ds(cb * CE * 8, CE * 8)],
                            r_v.at[pl.ds(0, CE * 8)])
            if want_deg:
                pltpu.sync_copy(dst_hbm.at[pl.ds(cb * CE, CE)],
                                dst_v.at[pl.ds(0, CE)])
            pltpu.async_copy(x_hbm.at[src_v], rows_v, sem)

        def tile_body(i, _):
            t = wid + i * NW
            sv = starts_v[pl.ds(t, 16)]
            s = sv[0]
            e = sv[1]
            cb0 = s // CE
            cb1 = (e + (CE - 1)) // CE

            @pl.when(i > 0)
            def _():
                pltpu.make_async_copy(a_v, _out_view(0), osem).wait()

            @pl.when(cb0 < cb1)
            def _():
                meta_and_gather(cb0)

            if tiled:
                def zbody(z, _):
                    for j in range(J):
                        a_v[z, pl.ds(16 * j, 16)] = jnp.zeros((16,),
                                                              jnp.float32)
                    return 0

                lax.fori_loop(0, ROWS, zbody, 0)
            else:
                def zbody(z, _):
                    for u in range(8):
                        a_v[pl.ds((z * 8 + u) * 16, 16)] = jnp.zeros(
                            (16,), jnp.float32)
                    return 0

                lax.fori_loop(0, ROWS * J // 8, zbody, 0)
            if want_deg:
                deg_v[...] = jnp.zeros((16,), jnp.float32)

            def chunk_body(cb, _):
                @pl.when(cb > cb0)
                def _():
                    meta_and_gather(cb)
                pltpu.make_async_copy(x_hbm.at[src_v], rows_v, sem).wait()
                elo = jnp.maximum(s - cb * CE, 0)
                ehi = jnp.minimum(e - cb * CE, CE)

                def edge_body(el, _):
                    segs = [rows_v[el, pl.ds(16 * j, 16)] for j in range(J)]
                    wv = w_v[pl.ds(el * 8, 16)]
                    rv = r_v[pl.ds(el * 8, 16)]
                    for c in range(8):
                        w = wv[c]
                        r = rv[c]
                        if tiled:
                            for j in range(J):
                                plsc.addupdate(
                                    a_v.at[r, pl.ds(16 * j, 16)],
                                    w * segs[j])
                        else:
                            base = r * ic
                            for j in range(J):
                                plsc.addupdate(
                                    a_v.at[pl.ds(base + 16 * j, 16)],
                                    w * segs[j])
                    if want_deg:
                        d = dst_v[pl.ds(el, 16)][0]
                        plsc.addupdate(
                            deg_v.at[pl.ds(0, 16)],
                            jnp.where(iota == d % TN, 1.0, 0.0).astype(
                                jnp.float32))
                    return 0

                lax.fori_loop(elo, ehi, edge_body, 0)
                return 0

            lax.fori_loop(cb0, cb1, chunk_body, 0)
            pltpu.async_copy(a_v, _out_view(t), osem)
            if want_deg:
                pltpu.sync_copy(deg_v, deg_hbm.at[pl.ds(t * 16, 16)])
            return 0

        lax.fori_loop(0, ntw, tile_body, 0)

        @pl.when(ntw > 0)
        def _():
            pltpu.make_async_copy(a_v, _out_view(0), osem).wait()

    res = k(x, src_s, w8f, r8f, dst_s, starts_p)
    return res if want_deg else res[0]


def _mm_layer(a_flat, W, x, root, bias, invdeg):
    n, ic = x.shape
    kd = W.shape[0] * W.shape[1]
    oc = W.shape[2]
    a2 = a_flat.reshape(n, kd)
    wr = W.reshape(kd, oc)
    kt = min(kd, 2048)
    nk = kd // kt
    nt = 1000
    ni = n // nt

    def body(a_ref, w_ref, x_ref, root_ref, bias_ref, invdeg_ref, out_ref,
             acc_ref):
        kk = pl.program_id(1)

        @pl.when(kk == 0)
        def _():
            acc_ref[...] = jnp.zeros_like(acc_ref)

        acc_ref[...] += jnp.dot(a_ref[...], w_ref[...],
                                preferred_element_type=jnp.float32)

        @pl.when(kk == nk - 1)
        def _():
            h = (acc_ref[...] * invdeg_ref[...]
                 + jnp.dot(x_ref[...], root_ref[...],
                           preferred_element_type=jnp.float32)
                 + bias_ref[...])
            out_ref[...] = jnp.where(h > 0, h, jnp.exp(h) - 1.0)

    return pl.pallas_call(
        body,
        grid=(ni, nk),
        in_specs=[
            pl.BlockSpec((nt, kt), lambda i, k: (i, k)),
            pl.BlockSpec((kt, oc), lambda i, k: (k, 0)),
            pl.BlockSpec((nt, ic), lambda i, k: (i, 0)),
            pl.BlockSpec((ic, oc), lambda i, k: (0, 0)),
            pl.BlockSpec((1, oc), lambda i, k: (0, 0)),
            pl.BlockSpec((nt, 1), lambda i, k: (i, 0)),
        ],
        out_specs=pl.BlockSpec((nt, oc), lambda i, k: (i, 0)),
        out_shape=jax.ShapeDtypeStruct((n, oc), jnp.float32),
        scratch_shapes=[pltpu.VMEM((nt, oc), jnp.float32)],
    )(a2, wr, x, root.reshape(ic, oc), bias.reshape(1, oc), invdeg)


def _mm_layer_t(a2d, W, x, root, bias, invdeg):
    n, ic = x.shape
    oc = W.shape[2]
    a3 = a2d.reshape(n, 64, ic)
    kb = 8
    nk = 64 // kb
    nt = 1000
    ni = n // nt

    def body(a_ref, w_ref, x_ref, root_ref, bias_ref, invdeg_ref, out_ref,
             acc_ref):
        kk = pl.program_id(1)

        @pl.when(kk == 0)
        def _():
            acc_ref[...] = jnp.zeros_like(acc_ref)

        for u in range(kb):
            acc_ref[...] += jnp.dot(a_ref[:, u, :], w_ref[u],
                                    preferred_element_type=jnp.float32)

        @pl.when(kk == nk - 1)
        def _():
            h = (acc_ref[...] * invdeg_ref[...]
                 + jnp.dot(x_ref[...], root_ref[...],
                           preferred_element_type=jnp.float32)
                 + bias_ref[...])
            out_ref[...] = jnp.where(h > 0, h, jnp.exp(h) - 1.0)

    return pl.pallas_call(
        body,
        grid=(ni, nk),
        in_specs=[
            pl.BlockSpec((nt, kb, ic), lambda i, k: (i, k, 0)),
            pl.BlockSpec((kb, ic, oc), lambda i, k: (k, 0, 0)),
            pl.BlockSpec((nt, ic), lambda i, k: (i, 0)),
            pl.BlockSpec((ic, oc), lambda i, k: (0, 0)),
            pl.BlockSpec((1, oc), lambda i, k: (0, 0)),
            pl.BlockSpec((nt, 1), lambda i, k: (i, 0)),
        ],
        out_specs=pl.BlockSpec((nt, oc), lambda i, k: (i, 0)),
        out_shape=jax.ShapeDtypeStruct((n, oc), jnp.float32),
        scratch_shapes=[pltpu.VMEM((nt, oc), jnp.float32)],
    )(a3, W, x, root.reshape(ic, oc), bias.reshape(1, oc), invdeg)


def _bn(h, gamma, beta):
    n, oc = h.shape
    ct = min(oc, 128)

    def body(h_ref, g_ref, b_ref, out_ref):
        hb = h_ref[...]
        m = jnp.mean(hb, axis=0, keepdims=True)
        v = jnp.mean((hb - m) ** 2, axis=0, keepdims=True)
        out_ref[...] = g_ref[...] * (hb - m) / jnp.sqrt(v + 1e-5) + b_ref[...]

    return pl.pallas_call(
        body,
        grid=(oc // ct,),
        in_specs=[
            pl.BlockSpec((n, ct), lambda j: (0, j)),
            pl.BlockSpec((1, ct), lambda j: (0, j)),
            pl.BlockSpec((1, ct), lambda j: (0, j)),
        ],
        out_specs=pl.BlockSpec((n, ct), lambda j: (0, j)),
        out_shape=jax.ShapeDtypeStruct((n, oc), jnp.float32),
    )(h, gamma.reshape(1, oc), beta.reshape(1, oc))


def _pool_max(h, cluster):
    n, oc = h.shape
    ct = 128

    def body(h_ref, cl_ref, out_ref):
        hb = h_ref[...]
        cl = cl_ref[...]
        rows = [jnp.max(jnp.where(cl == c, hb, -jnp.inf), axis=0,
                        keepdims=True) for c in range(4)]
        out_ref[...] = jnp.concatenate(rows + rows, axis=0)

    return pl.pallas_call(
        body,
        grid=(oc // ct,),
        in_specs=[
            pl.BlockSpec((n, ct), lambda j: (0, j)),
            pl.BlockSpec((n, 1), lambda j: (0, 0)),
        ],
        out_specs=pl.BlockSpec((8, ct), lambda j: (0, j)),
        out_shape=jax.ShapeDtypeStruct((8, oc), jnp.float32),
    )(h, cluster.reshape(n, 1))


def _head(feat, lin1, lin2):
    def body(feat_ref, l1_ref, l2_ref, out_ref):
        t = jnp.dot(feat_ref[...], l1_ref[...],
                    preferred_element_type=jnp.float32)
        out_ref[...] = jnp.dot(t, l2_ref[...],
                               preferred_element_type=jnp.float32)

    return pl.pallas_call(
        body,
        out_shape=jax.ShapeDtypeStruct((1, lin2.shape[1]), jnp.float32),
    )(feat, lin1, lin2)


def kernel(x, edge_attr, W1, root1, bias1, g1, be1, W2, root2, bias2, g2, be2,
           W3, root3, bias3, g3, be3, W4, root4, bias4, g4, be4,
           Wm, rootm, biasm, gm, bem, lin1, lin2, edge_index, pos):
    src = edge_index[0]
    dst = edge_index[1]
    n = x.shape[0]
    n_edges = src.shape[0]
    G = n // TN

    dst_s, src_s, ea0, ea1, ea2 = lax.sort(
        (dst.astype(jnp.int32), src.astype(jnp.int32),
         edge_attr[:, 0], edge_attr[:, 1], edge_attr[:, 2]),
        dimension=0, is_stable=False, num_keys=1)
    w8, k8 = _basis(jnp.stack([ea0, ea1, ea2], axis=1))
    w8f = w8.reshape(-1)
    r8f = ((dst_s % TN)[:, None] * 64 + k8).reshape(-1).astype(jnp.int32)
    starts = jnp.searchsorted(dst_s, jnp.arange(G + 1, dtype=jnp.int32) * TN
                              ).astype(jnp.int32)
    gp = ((G + 16 + 7) // 8) * 8
    starts_p = jnp.concatenate(
        [starts, jnp.full((gp - G - 1,), n_edges, jnp.int32)])

    x16 = jnp.concatenate([x, jnp.zeros((n, 15), jnp.float32)], axis=1)
    a1p, degp = _sc_layer(x16, src_s, w8f, r8f, dst_s, starts_p, 16,
                          want_deg=True)
    deg = degp.reshape(G, 16)[:, :TN].reshape(n, 1)
    invdeg = 1.0 / jnp.maximum(deg, 1.0)
    w1p = jnp.pad(W1, ((0, 0), (0, 15), (0, 0)))
    h = _mm_layer(a1p, w1p, x, root1, bias1, invdeg)
    h = _bn(h, g1, be1)
    for (W, root, bias, gm_, be_) in ((W2, root2, bias2, g2, be2),
                                      (W3, root3, bias3, g3, be3),
                                      (W4, root4, bias4, g4, be4)):
        a = _sc_layer(h, src_s, w8f, r8f, dst_s, starts_p, h.shape[1])
        if h.shape[1] >= 128:
            h = _mm_layer_t(a, W, h, root, bias, invdeg)
        else:
            h = _mm_layer(a, W, h, root, bias, invdeg)
        h = _bn(h, gm_, be_)

    cx = jnp.clip(jnp.floor(pos[:, 0] / 120.0), 0, 1).astype(jnp.int32)
    cy = jnp.clip(jnp.floor(pos[:, 1] / 90.0), 0, 1).astype(jnp.int32)
    cluster = cy * 2 + cx
    pooled = _pool_max(h, cluster)
    feat = pooled[:4].reshape(1, 4 * 512)
    labels = _head(feat, lin1, lin2)

    xm = jnp.zeros((n, 1), jnp.float32)
    return labels, xm
